# Initial kernel scaffold; baseline (speedup 1.0000x reference)
import jax
import jax.numpy as jnp
from jax.experimental import pallas as pl


def kernel(x, edge_index, enc_W0, enc_b0, enc_W1, enc_b1, enc_W2, enc_b2, enc_W3, enc_b3, dec_W0, dec_b0, dec_W1, dec_b1, dec_W2, dec_b2, dec_W3, dec_b3):
    raise NotImplementedError("write your pallas kernel here")


# speedup vs baseline: 2.9286x; 2.9286x over previous
import functools

import jax
import jax.numpy as jnp
from jax import lax
from jax.experimental import pallas as pl
from jax.experimental.pallas import tpu as pltpu
from jax.experimental.pallas import tpu_sc as plsc

N = 10000
E = 160000
D_IN = 256
D_HID = 512

NPAD = 10240
NW = 32
BLK = 128
NBLK = (E + NW * BLK - 1) // (NW * BLK)
EPAD = NW * NBLK * BLK
ROWS_PER_TILE = NPAD // 16

@functools.lru_cache(maxsize=1)
def _mesh():
    return plsc.VectorSubcoreMesh(core_axis_name="c", subcore_axis_name="s")


def _deg_body(dst_hbm, ones_hbm, zeros_hbm, out_hbm,
              dst_v, ones_v, zeros_v, acc_sh, sem):
    c = lax.axis_index("c")
    s = lax.axis_index("s")
    w = s * 2 + c
    pltpu.sync_copy(dst_hbm.at[w], dst_v)
    pltpu.sync_copy(ones_hbm, ones_v)
    pltpu.sync_copy(zeros_hbm, zeros_v)
    base = s * ROWS_PER_TILE
    for i in range(ROWS_PER_TILE // BLK):
        pltpu.sync_copy(zeros_v, acc_sh.at[pl.ds(base + i * BLK, BLK)])
    plsc.subcore_barrier()

    def body(j, carry):
        pltpu.sync_copy(ones_v, acc_sh.at[dst_v.at[j]], add=True)
        return carry

    lax.fori_loop(0, NBLK, body, 0)
    plsc.subcore_barrier()
    for i in range(ROWS_PER_TILE // BLK):
        pltpu.sync_copy(acc_sh.at[pl.ds(base + i * BLK, BLK)], zeros_v)
        pltpu.sync_copy(zeros_v, out_hbm.at[c, pl.ds(base + i * BLK, BLK)])


def _deg_kernel(dst3, ones128, zeros128):
    return pl.kernel(
        _deg_body,
        out_type=jax.ShapeDtypeStruct((2, NPAD, 128), jnp.float32),
        mesh=_mesh(),
        scratch_types=[
            pltpu.VMEM((NBLK, BLK), jnp.int32),
            pltpu.VMEM((BLK, 128), jnp.float32),
            pltpu.VMEM((BLK, 128), jnp.float32),
            pltpu.VMEM_SHARED((NPAD, 128), jnp.float32),
            pltpu.SemaphoreType.DMA,
        ],
    )(dst3, ones128, zeros128)


def _agg_body(nch, y_hbm, src_hbm, dst_hbm, zeros_hbm, out_hbm,
              src_v, dst_v, rows_v, zeros_v, acc_sh, sem):
    c = lax.axis_index("c")
    s = lax.axis_index("s")
    w = s * 2 + c
    pltpu.sync_copy(src_hbm.at[w], src_v)
    pltpu.sync_copy(dst_hbm.at[w], dst_v)
    pltpu.sync_copy(zeros_hbm, zeros_v)
    base = s * ROWS_PER_TILE
    for ch in range(nch):
        for i in range(ROWS_PER_TILE // BLK):
            pltpu.sync_copy(zeros_v, acc_sh.at[pl.ds(base + i * BLK, BLK)])
        plsc.subcore_barrier()

        def body(j, carry):
            pltpu.async_copy(y_hbm.at[ch].at[src_v.at[j]], rows_v, sem).wait()
            pltpu.sync_copy(rows_v, acc_sh.at[dst_v.at[j]], add=True)
            return carry

        lax.fori_loop(0, NBLK, body, 0)
        plsc.subcore_barrier()
        for i in range(ROWS_PER_TILE // BLK):
            pltpu.sync_copy(acc_sh.at[pl.ds(base + i * BLK, BLK)], rows_v)
            pltpu.sync_copy(
                rows_v, out_hbm.at[ch, c, pl.ds(base + i * BLK, BLK)])


def _agg_kernel(nch, y, src3, dst3, zeros128):
    return pl.kernel(
        functools.partial(_agg_body, nch),
        out_type=jax.ShapeDtypeStruct((nch, 2, NPAD, 128), jnp.float32),
        mesh=_mesh(),
        scratch_types=[
            pltpu.VMEM((NBLK, BLK), jnp.int32),
            pltpu.VMEM((NBLK, BLK), jnp.int32),
            pltpu.VMEM((BLK, 128), jnp.float32),
            pltpu.VMEM((BLK, 128), jnp.float32),
            pltpu.VMEM_SHARED((NPAD, 128), jnp.float32),
            pltpu.SemaphoreType.DMA,
        ],
    )(y, src3, dst3, zeros128)


_R = 256
_GRID = NPAD // _R


def _first_body(x_ref, w_ref, degp_ref, y_ref, dinv_ref):
    deg = 1.0 + degp_ref[0] + degp_ref[1]
    dinv = lax.rsqrt(deg)
    h = jnp.dot(x_ref[...], w_ref[...],
                preferred_element_type=jnp.float32,
                precision=lax.Precision.HIGHEST)
    for ch in range(4):
        y_ref[ch] = dinv * h[:, ch * 128:(ch + 1) * 128]
    dinv_ref[...] = dinv


def _first_kernel(x, w0, degp):
    return pl.pallas_call(
        _first_body,
        grid=(_GRID,),
        in_specs=[
            pl.BlockSpec((_R, D_IN), lambda m: (m, 0)),
            pl.BlockSpec((D_IN, D_HID), lambda m: (0, 0)),
            pl.BlockSpec((2, _R, 128), lambda m: (0, m, 0)),
        ],
        out_specs=[
            pl.BlockSpec((4, _R, 128), lambda m: (0, m, 0)),
            pl.BlockSpec((_R, 128), lambda m: (m, 0)),
        ],
        out_shape=[
            jax.ShapeDtypeStruct((4, NPAD, 128), jnp.float32),
            jax.ShapeDtypeStruct((NPAD, 128), jnp.float32),
        ],
    )(x, w0, degp)


def _mid_body(nch_in, nch_out, relu,
              p_ref, y_ref, dinv_ref, b_ref, w_ref, out_ref):
    dinv = dinv_ref[...]
    cols = []
    for ch in range(nch_in):
        t = p_ref[ch, 0] + p_ref[ch, 1] + y_ref[ch]
        t = dinv * t + b_ref[0, ch * 128:(ch + 1) * 128]
        cols.append(t)
    xb = jnp.concatenate(cols, axis=1)
    if relu:
        xb = jnp.maximum(xb, 0.0)
    h = jnp.dot(xb, w_ref[...],
                preferred_element_type=jnp.float32,
                precision=lax.Precision.HIGHEST)
    for ch in range(nch_out):
        out_ref[ch] = dinv * h[:, ch * 128:(ch + 1) * 128]


def _mid_kernel(nch_in, nch_out, relu, p, y, dinv, b, w):
    d_in = nch_in * 128
    d_out = nch_out * 128
    return pl.pallas_call(
        functools.partial(_mid_body, nch_in, nch_out, relu),
        grid=(_GRID,),
        in_specs=[
            pl.BlockSpec((nch_in, 2, _R, 128), lambda m: (0, 0, m, 0)),
            pl.BlockSpec((nch_in, _R, 128), lambda m: (0, m, 0)),
            pl.BlockSpec((_R, 128), lambda m: (m, 0)),
            pl.BlockSpec((1, d_in), lambda m: (0, 0)),
            pl.BlockSpec((d_in, d_out), lambda m: (0, 0)),
        ],
        out_specs=pl.BlockSpec((nch_out, _R, 128), lambda m: (0, m, 0)),
        out_shape=jax.ShapeDtypeStruct((nch_out, NPAD, 128), jnp.float32),
    )(p, y, dinv, b, w)


def _final_body(p_ref, y_ref, dinv_ref, b_ref, out_ref):
    dinv = dinv_ref[...]
    for ch in range(2):
        t = p_ref[ch, 0] + p_ref[ch, 1] + y_ref[ch]
        out_ref[:, ch * 128:(ch + 1) * 128] = (
            dinv * t + b_ref[0, ch * 128:(ch + 1) * 128])


def _final_kernel(p, y, dinv, b):
    return pl.pallas_call(
        _final_body,
        grid=(_GRID,),
        in_specs=[
            pl.BlockSpec((2, 2, _R, 128), lambda m: (0, 0, m, 0)),
            pl.BlockSpec((2, _R, 128), lambda m: (0, m, 0)),
            pl.BlockSpec((_R, 128), lambda m: (m, 0)),
            pl.BlockSpec((1, D_IN), lambda m: (0, 0)),
        ],
        out_specs=pl.BlockSpec((_R, D_IN), lambda m: (m, 0)),
        out_shape=jax.ShapeDtypeStruct((NPAD, D_IN), jnp.float32),
    )(p, y, dinv, b)


def kernel(x, edge_index,
           enc_W0, enc_b0, enc_W1, enc_b1, enc_W2, enc_b2, enc_W3, enc_b3,
           dec_W0, dec_b0, dec_W1, dec_b1, dec_W2, dec_b2, dec_W3, dec_b3):
    src = edge_index[0]
    dst = edge_index[1]
    pad = EPAD - E
    src3 = jnp.concatenate(
        [src, jnp.zeros((pad,), jnp.int32)]).reshape(NW, NBLK, BLK)
    dst3 = jnp.concatenate(
        [dst, jnp.full((pad,), N, jnp.int32)]).reshape(NW, NBLK, BLK)
    xp = jnp.pad(x, ((0, NPAD - N), (0, 0)))
    ones128 = jnp.ones((BLK, 128), jnp.float32)
    zeros128 = jnp.zeros((BLK, 128), jnp.float32)

    degp = _deg_kernel(dst3, ones128, zeros128)
    y, dinv = _first_kernel(xp, enc_W0, degp)

    mids = [(enc_b0, enc_W1, True), (enc_b1, enc_W2, True),
            (enc_b2, enc_W3, True), (enc_b3, dec_W0, False),
            (dec_b0, dec_W1, True), (dec_b1, dec_W2, True),
            (dec_b2, dec_W3, True)]
    for b_prev, w_cur, relu in mids:
        nch_out = w_cur.shape[1] // 128
        p = _agg_kernel(4, y, src3, dst3, zeros128)
        y = _mid_kernel(4, nch_out, relu, p, y, dinv,
                        b_prev.reshape(1, -1), w_cur)

    p = _agg_kernel(2, y, src3, dst3, zeros128)
    out = _final_kernel(p, y, dinv, dec_b3.reshape(1, -1))
    return out[:N]


# speedup vs baseline: 3.4304x; 1.1714x over previous
import functools

import jax
import jax.numpy as jnp
from jax import lax
from jax.experimental import pallas as pl
from jax.experimental.pallas import tpu as pltpu
from jax.experimental.pallas import tpu_sc as plsc

N = 10000
E = 160000
D_IN = 256
D_HID = 512

NPAD = 10240
NW = 32
BLK = 128
NBLK = (E + NW * BLK - 1) // (NW * BLK)
EPAD = NW * NBLK * BLK
ROWS_PER_TILE = NPAD // 16

@functools.lru_cache(maxsize=1)
def _mesh():
    return plsc.VectorSubcoreMesh(core_axis_name="c", subcore_axis_name="s")


_SGRP = 10


def _deg_body(dst_hbm, ones_hbm, zeros_hbm, out_hbm,
              dst_v, ones_v, acc_sh, sem):
    c = lax.axis_index("c")
    s = lax.axis_index("s")
    w = s * 2 + c
    pltpu.sync_copy(dst_hbm.at[w], dst_v)
    pltpu.sync_copy(ones_hbm, ones_v)
    base = s * ROWS_PER_TILE
    pltpu.sync_copy(zeros_hbm, acc_sh.at[pl.ds(base, ROWS_PER_TILE)])
    plsc.subcore_barrier()

    def body(g, carry):
        j0 = g * _SGRP

        def fire(j, carry2):
            pltpu.async_copy(ones_v, acc_sh.at[dst_v.at[j0 + j]], sem,
                             add=True)
            return carry2

        lax.fori_loop(0, _SGRP, fire, 0)

        def drain(j, carry2):
            pltpu.make_async_copy(
                ones_v, acc_sh.at[dst_v.at[0]], sem).wait()
            return carry2

        lax.fori_loop(0, _SGRP, drain, 0)
        return carry

    lax.fori_loop(0, NBLK // _SGRP, body, 0)
    plsc.subcore_barrier()
    for i in range(ROWS_PER_TILE // BLK):
        pltpu.sync_copy(acc_sh.at[pl.ds(base + i * BLK, BLK)], ones_v)
        pltpu.sync_copy(ones_v, out_hbm.at[c, pl.ds(base + i * BLK, BLK)])


def _deg_kernel(dst3, ones128, zeros640):
    return pl.kernel(
        _deg_body,
        out_type=jax.ShapeDtypeStruct((2, NPAD, 128), jnp.float32),
        mesh=_mesh(),
        scratch_types=[
            pltpu.VMEM((NBLK, BLK), jnp.int32),
            pltpu.VMEM((BLK, 128), jnp.float32),
            pltpu.VMEM_SHARED((NPAD, 128), jnp.float32),
            pltpu.SemaphoreType.DMA,
        ],
    )(dst3, ones128, zeros640)


def _agg_body(nch, y_hbm, src_hbm, dst_hbm, zeros_hbm, out_hbm,
              src_v, dst_v, rows0, rows1, acc_sh, sem0, sem1):
    c = lax.axis_index("c")
    s = lax.axis_index("s")
    w = s * 2 + c
    pltpu.sync_copy(src_hbm.at[w], src_v)
    pltpu.sync_copy(dst_hbm.at[w], dst_v)
    base = s * ROWS_PER_TILE
    for ch in range(nch):
        pltpu.sync_copy(zeros_hbm, acc_sh.at[pl.ds(base, ROWS_PER_TILE)])
        plsc.subcore_barrier()
        pltpu.async_copy(y_hbm.at[ch].at[src_v.at[0]], rows0, sem0)

        def body(g, carry):
            j0 = 2 * g
            pltpu.async_copy(y_hbm.at[ch].at[src_v.at[j0 + 1]], rows1, sem1)
            pltpu.make_async_copy(
                y_hbm.at[ch].at[src_v.at[j0]], rows0, sem0).wait()
            pltpu.sync_copy(rows0, acc_sh.at[dst_v.at[j0]], add=True)

            @pl.when(j0 + 2 < NBLK)
            def _():
                pltpu.async_copy(
                    y_hbm.at[ch].at[src_v.at[j0 + 2]], rows0, sem0)

            pltpu.make_async_copy(
                y_hbm.at[ch].at[src_v.at[j0 + 1]], rows1, sem1).wait()
            pltpu.sync_copy(rows1, acc_sh.at[dst_v.at[j0 + 1]], add=True)
            return carry

        lax.fori_loop(0, NBLK // 2, body, 0)
        plsc.subcore_barrier()
        for i in range(ROWS_PER_TILE // BLK):
            pltpu.sync_copy(acc_sh.at[pl.ds(base + i * BLK, BLK)], rows0)
            pltpu.sync_copy(
                rows0, out_hbm.at[ch, c, pl.ds(base + i * BLK, BLK)])


def _agg_kernel(nch, y, src3, dst3, zeros640):
    return pl.kernel(
        functools.partial(_agg_body, nch),
        out_type=jax.ShapeDtypeStruct((nch, 2, NPAD, 128), jnp.float32),
        mesh=_mesh(),
        scratch_types=[
            pltpu.VMEM((NBLK, BLK), jnp.int32),
            pltpu.VMEM((NBLK, BLK), jnp.int32),
            pltpu.VMEM((BLK, 128), jnp.float32),
            pltpu.VMEM((BLK, 128), jnp.float32),
            pltpu.VMEM_SHARED((NPAD, 128), jnp.float32),
            pltpu.SemaphoreType.DMA,
            pltpu.SemaphoreType.DMA,
        ],
    )(y, src3, dst3, zeros640)


_R = 256
_GRID = NPAD // _R


def _first_body(x_ref, w_ref, degp_ref, y_ref, dinv_ref):
    deg = 1.0 + degp_ref[0] + degp_ref[1]
    dinv = lax.rsqrt(deg)
    h = jnp.dot(x_ref[...], w_ref[...],
                preferred_element_type=jnp.float32,
                precision=lax.Precision.HIGHEST)
    for ch in range(4):
        y_ref[ch] = dinv * h[:, ch * 128:(ch + 1) * 128]
    dinv_ref[...] = dinv


def _first_kernel(x, w0, degp):
    return pl.pallas_call(
        _first_body,
        grid=(_GRID,),
        in_specs=[
            pl.BlockSpec((_R, D_IN), lambda m: (m, 0)),
            pl.BlockSpec((D_IN, D_HID), lambda m: (0, 0)),
            pl.BlockSpec((2, _R, 128), lambda m: (0, m, 0)),
        ],
        out_specs=[
            pl.BlockSpec((4, _R, 128), lambda m: (0, m, 0)),
            pl.BlockSpec((_R, 128), lambda m: (m, 0)),
        ],
        out_shape=[
            jax.ShapeDtypeStruct((4, NPAD, 128), jnp.float32),
            jax.ShapeDtypeStruct((NPAD, 128), jnp.float32),
        ],
    )(x, w0, degp)


def _mid_body(nch_in, nch_out, relu,
              p_ref, y_ref, dinv_ref, b_ref, w_ref, out_ref):
    dinv = dinv_ref[...]
    cols = []
    for ch in range(nch_in):
        t = p_ref[ch, 0] + p_ref[ch, 1] + y_ref[ch]
        t = dinv * t + b_ref[0, ch * 128:(ch + 1) * 128]
        cols.append(t)
    xb = jnp.concatenate(cols, axis=1)
    if relu:
        xb = jnp.maximum(xb, 0.0)
    h = jnp.dot(xb, w_ref[...],
                preferred_element_type=jnp.float32,
                precision=lax.Precision.HIGHEST)
    for ch in range(nch_out):
        out_ref[ch] = dinv * h[:, ch * 128:(ch + 1) * 128]


def _mid_kernel(nch_in, nch_out, relu, p, y, dinv, b, w):
    d_in = nch_in * 128
    d_out = nch_out * 128
    return pl.pallas_call(
        functools.partial(_mid_body, nch_in, nch_out, relu),
        grid=(_GRID,),
        in_specs=[
            pl.BlockSpec((nch_in, 2, _R, 128), lambda m: (0, 0, m, 0)),
            pl.BlockSpec((nch_in, _R, 128), lambda m: (0, m, 0)),
            pl.BlockSpec((_R, 128), lambda m: (m, 0)),
            pl.BlockSpec((1, d_in), lambda m: (0, 0)),
            pl.BlockSpec((d_in, d_out), lambda m: (0, 0)),
        ],
        out_specs=pl.BlockSpec((nch_out, _R, 128), lambda m: (0, m, 0)),
        out_shape=jax.ShapeDtypeStruct((nch_out, NPAD, 128), jnp.float32),
    )(p, y, dinv, b, w)


def _final_body(p_ref, y_ref, dinv_ref, b_ref, out_ref):
    dinv = dinv_ref[...]
    for ch in range(2):
        t = p_ref[ch, 0] + p_ref[ch, 1] + y_ref[ch]
        out_ref[:, ch * 128:(ch + 1) * 128] = (
            dinv * t + b_ref[0, ch * 128:(ch + 1) * 128])


def _final_kernel(p, y, dinv, b):
    return pl.pallas_call(
        _final_body,
        grid=(_GRID,),
        in_specs=[
            pl.BlockSpec((2, 2, _R, 128), lambda m: (0, 0, m, 0)),
            pl.BlockSpec((2, _R, 128), lambda m: (0, m, 0)),
            pl.BlockSpec((_R, 128), lambda m: (m, 0)),
            pl.BlockSpec((1, D_IN), lambda m: (0, 0)),
        ],
        out_specs=pl.BlockSpec((_R, D_IN), lambda m: (m, 0)),
        out_shape=jax.ShapeDtypeStruct((NPAD, D_IN), jnp.float32),
    )(p, y, dinv, b)


def kernel(x, edge_index,
           enc_W0, enc_b0, enc_W1, enc_b1, enc_W2, enc_b2, enc_W3, enc_b3,
           dec_W0, dec_b0, dec_W1, dec_b1, dec_W2, dec_b2, dec_W3, dec_b3):
    src = edge_index[0]
    dst = edge_index[1]
    pad = EPAD - E
    src3 = jnp.concatenate(
        [src, jnp.zeros((pad,), jnp.int32)]).reshape(NW, NBLK, BLK)
    dst3 = jnp.concatenate(
        [dst, jnp.full((pad,), N, jnp.int32)]).reshape(NW, NBLK, BLK)
    xp = jnp.pad(x, ((0, NPAD - N), (0, 0)))
    ones128 = jnp.ones((BLK, 128), jnp.float32)
    zeros640 = jnp.zeros((ROWS_PER_TILE, 128), jnp.float32)

    degp = _deg_kernel(dst3, ones128, zeros640)
    y, dinv = _first_kernel(xp, enc_W0, degp)

    mids = [(enc_b0, enc_W1, True), (enc_b1, enc_W2, True),
            (enc_b2, enc_W3, True), (enc_b3, dec_W0, False),
            (dec_b0, dec_W1, True), (dec_b1, dec_W2, True),
            (dec_b2, dec_W3, True)]
    for b_prev, w_cur, relu in mids:
        nch_out = w_cur.shape[1] // 128
        p = _agg_kernel(4, y, src3, dst3, zeros640)
        y = _mid_kernel(4, nch_out, relu, p, y, dinv,
                        b_prev.reshape(1, -1), w_cur)

    p = _agg_kernel(2, y, src3, dst3, zeros640)
    out = _final_kernel(p, y, dinv, dec_b3.reshape(1, -1))
    return out[:N]


# speedup vs baseline: 5.6741x; 1.6541x over previous
import functools

import jax
import jax.numpy as jnp
from jax import lax
from jax.experimental import pallas as pl
from jax.experimental.pallas import tpu as pltpu
from jax.experimental.pallas import tpu_sc as plsc

N = 10000
E = 160000
D_IN = 256
D_HID = 512

NPAD = 10240
NW = 32
BLK = 64
NBLK = (E + NW * BLK - 1) // (NW * BLK)
EPAD = NW * NBLK * BLK
ROWS_PER_TILE = NPAD // 16

@functools.lru_cache(maxsize=1)
def _mesh():
    return plsc.VectorSubcoreMesh(core_axis_name="c", subcore_axis_name="s")


_SGRP = 10


def _deg_body(dst_hbm, ones_hbm, zeros_hbm, out_hbm,
              dst_v, ones_v, acc_sh, sem):
    c = lax.axis_index("c")
    s = lax.axis_index("s")
    w = s * 2 + c
    pltpu.sync_copy(dst_hbm.at[w], dst_v)
    pltpu.sync_copy(ones_hbm, ones_v)
    base = s * ROWS_PER_TILE
    pltpu.sync_copy(zeros_hbm, acc_sh.at[pl.ds(base, ROWS_PER_TILE)])
    plsc.subcore_barrier()

    def body(g, carry):
        j0 = g * _SGRP

        def fire(j, carry2):
            pltpu.async_copy(ones_v, acc_sh.at[dst_v.at[j0 + j]], sem,
                             add=True)
            return carry2

        lax.fori_loop(0, _SGRP, fire, 0)

        def drain(j, carry2):
            pltpu.make_async_copy(
                ones_v, acc_sh.at[dst_v.at[0]], sem).wait()
            return carry2

        lax.fori_loop(0, _SGRP, drain, 0)
        return carry

    lax.fori_loop(0, NBLK // _SGRP, body, 0)
    plsc.subcore_barrier()
    pltpu.sync_copy(acc_sh.at[pl.ds(base, ROWS_PER_TILE)],
                    out_hbm.at[c, pl.ds(base, ROWS_PER_TILE)])


def _deg_kernel(dst3, ones_blk, zeros640):
    return pl.kernel(
        _deg_body,
        out_type=jax.ShapeDtypeStruct((2, NPAD, 128), jnp.float32),
        mesh=_mesh(),
        scratch_types=[
            pltpu.VMEM((NBLK, BLK), jnp.int32),
            pltpu.VMEM((BLK, 128), jnp.float32),
            pltpu.VMEM_SHARED((NPAD, 128), jnp.float32),
            pltpu.SemaphoreType.DMA,
        ],
    )(dst3, ones_blk, zeros640)


def _agg_body(nch, y_hbm, src_hbm, dst_hbm, zeros_hbm, out_hbm,
              src_v, dst_v, r0, r1, r2, acc_sh,
              g0, g1, g2, s0, s1, s2):
    rows = [r0, r1, r2]
    gs = [g0, g1, g2]
    ss = [s0, s1, s2]
    c = lax.axis_index("c")
    s = lax.axis_index("s")
    w = s * 2 + c
    pltpu.sync_copy(src_hbm.at[w], src_v)
    pltpu.sync_copy(dst_hbm.at[w], dst_v)
    base = s * ROWS_PER_TILE
    for ch in range(nch):
        pltpu.sync_copy(zeros_hbm, acc_sh.at[pl.ds(base, ROWS_PER_TILE)])
        plsc.subcore_barrier()

        def gfire(j, b):
            pltpu.async_copy(y_hbm.at[ch].at[src_v.at[j]], rows[b], gs[b])

        def gwait(j, b):
            pltpu.make_async_copy(
                y_hbm.at[ch].at[src_v.at[j]], rows[b], gs[b]).wait()

        def sfire(j, b):
            pltpu.async_copy(rows[b], acc_sh.at[dst_v.at[j]], ss[b],
                             add=True)

        def swait(b):
            pltpu.make_async_copy(
                rows[b], acc_sh.at[dst_v.at[0]], ss[b]).wait()

        gfire(0, 0)
        gfire(1, 1)
        gwait(0, 0)
        sfire(0, 0)
        gfire(2, 2)
        gwait(1, 1)
        sfire(1, 1)
        swait(0)
        gfire(3, 0)
        gwait(2, 2)
        sfire(2, 2)
        swait(1)
        gfire(4, 1)

        def body(g, carry):
            j0 = 3 * g
            for b in range(3):
                j = j0 + b
                gwait(j, b)
                sfire(j, b)
                b2 = (b + 2) % 3
                swait(b2)
                gfire(j + 2, b2)
            return carry

        lax.fori_loop(1, (NBLK - 2) // 3, body, 0)
        gwait(NBLK - 2, 0)
        sfire(NBLK - 2, 0)
        gwait(NBLK - 1, 1)
        sfire(NBLK - 1, 1)
        swait(0)
        swait(1)
        swait(2)
        plsc.subcore_barrier()
        pltpu.sync_copy(acc_sh.at[pl.ds(base, ROWS_PER_TILE)],
                        out_hbm.at[ch, c, pl.ds(base, ROWS_PER_TILE)])


def _agg_kernel(nch, y, src3, dst3, zeros640):
    return pl.kernel(
        functools.partial(_agg_body, nch),
        out_type=jax.ShapeDtypeStruct((nch, 2, NPAD, 128), jnp.float32),
        mesh=_mesh(),
        scratch_types=[
            pltpu.VMEM((NBLK, BLK), jnp.int32),
            pltpu.VMEM((NBLK, BLK), jnp.int32),
            pltpu.VMEM((BLK, 128), jnp.float32),
            pltpu.VMEM((BLK, 128), jnp.float32),
            pltpu.VMEM((BLK, 128), jnp.float32),
            pltpu.VMEM_SHARED((NPAD, 128), jnp.float32),
        ] + [pltpu.SemaphoreType.DMA] * 6,
    )(y, src3, dst3, zeros640)


_R = 256
_GRID = NPAD // _R


def _first_body(x_ref, w_ref, degp_ref, y_ref, dinv_ref):
    deg = 1.0 + degp_ref[0] + degp_ref[1]
    dinv = lax.rsqrt(deg)
    h = jnp.dot(x_ref[...], w_ref[...],
                preferred_element_type=jnp.float32,
                precision=lax.Precision.HIGHEST)
    for ch in range(4):
        y_ref[ch] = dinv * h[:, ch * 128:(ch + 1) * 128]
    dinv_ref[...] = dinv


def _first_kernel(x, w0, degp):
    return pl.pallas_call(
        _first_body,
        grid=(_GRID,),
        in_specs=[
            pl.BlockSpec((_R, D_IN), lambda m: (m, 0)),
            pl.BlockSpec((D_IN, D_HID), lambda m: (0, 0)),
            pl.BlockSpec((2, _R, 128), lambda m: (0, m, 0)),
        ],
        out_specs=[
            pl.BlockSpec((4, _R, 128), lambda m: (0, m, 0)),
            pl.BlockSpec((_R, 128), lambda m: (m, 0)),
        ],
        out_shape=[
            jax.ShapeDtypeStruct((4, NPAD, 128), jnp.float32),
            jax.ShapeDtypeStruct((NPAD, 128), jnp.float32),
        ],
    )(x, w0, degp)


def _mid_body(nch_in, nch_out, relu,
              p_ref, y_ref, dinv_ref, b_ref, w_ref, out_ref):
    dinv = dinv_ref[...]
    cols = []
    for ch in range(nch_in):
        t = p_ref[ch, 0] + p_ref[ch, 1] + y_ref[ch]
        t = dinv * t + b_ref[0, ch * 128:(ch + 1) * 128]
        cols.append(t)
    xb = jnp.concatenate(cols, axis=1)
    if relu:
        xb = jnp.maximum(xb, 0.0)
    h = jnp.dot(xb, w_ref[...],
                preferred_element_type=jnp.float32,
                precision=lax.Precision.HIGHEST)
    for ch in range(nch_out):
        out_ref[ch] = dinv * h[:, ch * 128:(ch + 1) * 128]


def _mid_kernel(nch_in, nch_out, relu, p, y, dinv, b, w):
    d_in = nch_in * 128
    d_out = nch_out * 128
    return pl.pallas_call(
        functools.partial(_mid_body, nch_in, nch_out, relu),
        grid=(_GRID,),
        in_specs=[
            pl.BlockSpec((nch_in, 2, _R, 128), lambda m: (0, 0, m, 0)),
            pl.BlockSpec((nch_in, _R, 128), lambda m: (0, m, 0)),
            pl.BlockSpec((_R, 128), lambda m: (m, 0)),
            pl.BlockSpec((1, d_in), lambda m: (0, 0)),
            pl.BlockSpec((d_in, d_out), lambda m: (0, 0)),
        ],
        out_specs=pl.BlockSpec((nch_out, _R, 128), lambda m: (0, m, 0)),
        out_shape=jax.ShapeDtypeStruct((nch_out, NPAD, 128), jnp.float32),
    )(p, y, dinv, b, w)


def _final_body(p_ref, y_ref, dinv_ref, b_ref, out_ref):
    dinv = dinv_ref[...]
    for ch in range(2):
        t = p_ref[ch, 0] + p_ref[ch, 1] + y_ref[ch]
        out_ref[:, ch * 128:(ch + 1) * 128] = (
            dinv * t + b_ref[0, ch * 128:(ch + 1) * 128])


def _final_kernel(p, y, dinv, b):
    return pl.pallas_call(
        _final_body,
        grid=(_GRID,),
        in_specs=[
            pl.BlockSpec((2, 2, _R, 128), lambda m: (0, 0, m, 0)),
            pl.BlockSpec((2, _R, 128), lambda m: (0, m, 0)),
            pl.BlockSpec((_R, 128), lambda m: (m, 0)),
            pl.BlockSpec((1, D_IN), lambda m: (0, 0)),
        ],
        out_specs=pl.BlockSpec((_R, D_IN), lambda m: (m, 0)),
        out_shape=jax.ShapeDtypeStruct((NPAD, D_IN), jnp.float32),
    )(p, y, dinv, b)


def kernel(x, edge_index,
           enc_W0, enc_b0, enc_W1, enc_b1, enc_W2, enc_b2, enc_W3, enc_b3,
           dec_W0, dec_b0, dec_W1, dec_b1, dec_W2, dec_b2, dec_W3, dec_b3):
    src = edge_index[0]
    dst = edge_index[1]
    pad = EPAD - E
    src3 = jnp.concatenate(
        [src, jnp.zeros((pad,), jnp.int32)]).reshape(NW, NBLK, BLK)
    dst3 = jnp.concatenate(
        [dst, jnp.full((pad,), N, jnp.int32)]).reshape(NW, NBLK, BLK)
    xp = jnp.pad(x, ((0, NPAD - N), (0, 0)))
    ones128 = jnp.ones((BLK, 128), jnp.float32)
    zeros640 = jnp.zeros((ROWS_PER_TILE, 128), jnp.float32)

    degp = _deg_kernel(dst3, ones128, zeros640)
    y, dinv = _first_kernel(xp, enc_W0, degp)

    mids = [(enc_b0, enc_W1, True), (enc_b1, enc_W2, True),
            (enc_b2, enc_W3, True), (enc_b3, dec_W0, False),
            (dec_b0, dec_W1, True), (dec_b1, dec_W2, True),
            (dec_b2, dec_W3, True)]
    for b_prev, w_cur, relu in mids:
        nch_out = w_cur.shape[1] // 128
        p = _agg_kernel(4, y, src3, dst3, zeros640)
        y = _mid_kernel(4, nch_out, relu, p, y, dinv,
                        b_prev.reshape(1, -1), w_cur)

    p = _agg_kernel(2, y, src3, dst3, zeros640)
    out = _final_kernel(p, y, dinv, dec_b3.reshape(1, -1))
    return out[:N]
